# baseline (speedup 1.0000x reference)
import jax
import jax.numpy as jnp
from jax.experimental import pallas as pl
from jax.experimental.pallas import tpu as pltpu

N = 2048
BM = 256
NB = N // BM
NS = NB + 1

BF = jnp.bfloat16


def _leaky(x):
    return jnp.where(x >= 0, x, 0.01 * x)


def _dot(a, b):
    return jnp.dot(a, b, preferred_element_type=jnp.float32)


def _body(x_ref, wp0_ref, wg0_ref, wl0_ref, wp1_ref, wg1_ref, wl1_ref,
          wp2_ref, wg2_ref, wl2_ref, adj_ref, o_ref,
          xb_scr, h1_scr, h2_scr, q_scr, v_scr, u_scr, adjb_scr, lg_scr):
    t = pl.program_id(0)
    layer = t // NS
    j = t % NS

    def phase(h_scr, wp_ref, wg_ref, wl_ref, cin, cout, store_out, act,
              fill_h=None, cache_adj=False):
        @pl.when(j == 0)
        def _prep():
            if fill_h is not None:
                h_scr[...] = fill_h()
            h = h_scr[...]
            q_scr[:, :cin] = _dot(h, wp_ref[...].astype(BF)).astype(BF)
            v_scr[:, :cout] = _dot(h, wg_ref[...].astype(BF)).astype(BF)
            u_scr[:, :cout] = _dot(h, wl_ref[...].astype(BF)).astype(BF)

        if cache_adj:
            @pl.when(j >= 1)
            def _cache():
                adjb_scr[pl.ds((j - 1) * BM, BM), :] = (
                    adj_ref[...].astype(BF))

        @pl.when(j < NB)
        def _logits():
            q_j = q_scr[pl.ds(j * BM, BM), :cin]
            lg = jax.lax.dot_general(
                q_j, h_scr[...], (((1,), (1,)), ((), ())),
                preferred_element_type=jnp.float32)
            lg_scr[pl.ds((j % 2) * BM, BM), :] = lg

        @pl.when(j >= 1)
        def _finish():
            i = j - 1
            logits = lg_scr[pl.ds((i % 2) * BM, BM), :]
            m = jnp.max(logits, axis=1, keepdims=True)
            e = jnp.exp(logits - m)
            s = jnp.sum(e, axis=1, keepdims=True)
            og = _dot(e.astype(BF), v_scr[:, :cout]) / s
            ol = _dot(adjb_scr[pl.ds(i * BM, BM), :], u_scr[:, :cout])
            if act:
                out = _leaky(og) + _leaky(ol)
            else:
                out = og + ol
            store_out(out, i)

    @pl.when(layer == 0)
    def _l0():
        def store(out, i):
            h1_scr[pl.ds(i * BM, BM), :] = out.astype(BF)
        phase(xb_scr, wp0_ref, wg0_ref, wl0_ref, 256, 256, store, True,
              fill_h=lambda: x_ref[...].astype(BF), cache_adj=True)

    @pl.when(layer == 1)
    def _l1():
        def store(out, i):
            h2_scr[pl.ds(i * BM, BM), :] = out.astype(BF)
        phase(h1_scr, wp1_ref, wg1_ref, wl1_ref, 256, 512, store, True)

    @pl.when(layer == 2)
    def _l2():
        def store(out, i):
            o_ref[pl.ds(i * BM, BM), :] = out
        phase(h2_scr, wp2_ref, wg2_ref, wl2_ref, 512, 64, store, False)


def kernel(x, adj, Wp0, Wg0, Wl0, Wp1, Wg1, Wl1, Wp2, Wg2, Wl2):
    f32 = jnp.float32
    return pl.pallas_call(
        _body,
        grid=(3 * NS,),
        in_specs=[
            pl.BlockSpec((N, 256), lambda t: (0, 0)),
            pl.BlockSpec((256, 256), lambda t: (0, 0)),
            pl.BlockSpec((256, 256), lambda t: (0, 0)),
            pl.BlockSpec((256, 256), lambda t: (0, 0)),
            pl.BlockSpec((256, 256), lambda t: (0, 0)),
            pl.BlockSpec((256, 512), lambda t: (0, 0)),
            pl.BlockSpec((256, 512), lambda t: (0, 0)),
            pl.BlockSpec((512, 512), lambda t: (0, 0)),
            pl.BlockSpec((512, 64), lambda t: (0, 0)),
            pl.BlockSpec((512, 64), lambda t: (0, 0)),
            pl.BlockSpec((BM, N),
                         lambda t: (jnp.clip(t - 1, 0, NB - 1), 0)),
        ],
        out_specs=pl.BlockSpec((N, 64), lambda t: (0, 0)),
        out_shape=jax.ShapeDtypeStruct((N, 64), f32),
        scratch_shapes=[
            pltpu.VMEM((N, 256), BF),
            pltpu.VMEM((N, 256), BF),
            pltpu.VMEM((N, 512), BF),
            pltpu.VMEM((N, 512), BF),
            pltpu.VMEM((N, 512), BF),
            pltpu.VMEM((N, 512), BF),
            pltpu.VMEM((N, N), BF),
            pltpu.VMEM((2 * BM, N), f32),
        ],
    )(x, Wp0, Wg0, Wl0, Wp1, Wg1, Wl1, Wp2, Wg2, Wl2, adj)


# speedup vs baseline: 1.1122x; 1.1122x over previous
import jax
import jax.numpy as jnp
from jax.experimental import pallas as pl
from jax.experimental.pallas import tpu as pltpu

N = 2048
BM = 256
NB = N // BM

BF = jnp.bfloat16

LAYERS = ((256, 256, True), (256, 512, True), (512, 64, False))


def _leaky(x):
    return jnp.where(x >= 0, x, 0.01 * x)


def _dot(a, b):
    return jnp.dot(a, b, preferred_element_type=jnp.float32)


def _body(x_ref, wp0_ref, wg0_ref, wl0_ref, wp1_ref, wg1_ref, wl1_ref,
          wp2_ref, wg2_ref, wl2_ref, adj_ref, o_ref,
          xb_scr, h1_scr, h2_scr, q_scr, v_scr, u_scr, adjb_scr, lg_scr):
    t = pl.program_id(0)

    w = ((wp0_ref, wg0_ref, wl0_ref), (wp1_ref, wg1_ref, wl1_ref),
         (wp2_ref, wg2_ref, wl2_ref))
    hs = (xb_scr, h1_scr, h2_scr)

    @pl.when((t >= 1) & (t <= NB))
    def _cache():
        adjb_scr[pl.ds((t - 1) * BM, BM), :] = adj_ref[...].astype(BF)

    def prep(l):
        h = hs[l][...]
        wp, wg, wl = w[l]
        cin, cout, _ = LAYERS[l]
        q_scr[:, :cin] = _dot(h, wp[...].astype(BF)).astype(BF)
        v_scr[:, :cout] = _dot(h, wg[...].astype(BF)).astype(BF)
        u_scr[:, :cout] = _dot(h, wl[...].astype(BF)).astype(BF)

    def logits(l, i):
        cin = LAYERS[l][0]
        q_i = q_scr[pl.ds(i * BM, BM), :cin]
        lg_scr[...] = jax.lax.dot_general(
            q_i, hs[l][...], (((1,), (1,)), ((), ())),
            preferred_element_type=jnp.float32)

    def finish(l, i):
        cout, act = LAYERS[l][1:]
        logits_i = lg_scr[...]
        m = jnp.max(logits_i, axis=1, keepdims=True)
        e = jnp.exp(logits_i - m)
        s = jnp.sum(e, axis=1, keepdims=True)
        og = _dot(e.astype(BF), v_scr[:, :cout]) / s
        ol = _dot(adjb_scr[pl.ds(i * BM, BM), :], u_scr[:, :cout])
        if act:
            out = _leaky(og) + _leaky(ol)
        else:
            out = og + ol
        if l == 2:
            o_ref[pl.ds(i * BM, BM), :] = out
        else:
            hs[l + 1][pl.ds(i * BM, BM), :] = out.astype(BF)

    @pl.when(t == 0)
    def _start():
        xb_scr[...] = x_ref[...].astype(BF)
        prep(0)
        logits(0, 0)

    for l in range(3):
        if l > 0:
            @pl.when(t == l * NB)
            def _boundary(l=l):
                finish(l - 1, NB - 1)
                prep(l)
                logits(l, 0)

        @pl.when((t > l * NB) & (t < (l + 1) * NB))
        def _steady(l=l):
            i = t - l * NB
            finish(l, i - 1)
            logits(l, i)

    @pl.when(t == 3 * NB)
    def _tail():
        finish(2, NB - 1)


def kernel(x, adj, Wp0, Wg0, Wl0, Wp1, Wg1, Wl1, Wp2, Wg2, Wl2):
    f32 = jnp.float32
    return pl.pallas_call(
        _body,
        grid=(3 * NB + 1,),
        in_specs=[
            pl.BlockSpec((N, 256), lambda t: (0, 0)),
            pl.BlockSpec((256, 256), lambda t: (0, 0)),
            pl.BlockSpec((256, 256), lambda t: (0, 0)),
            pl.BlockSpec((256, 256), lambda t: (0, 0)),
            pl.BlockSpec((256, 256), lambda t: (0, 0)),
            pl.BlockSpec((256, 512), lambda t: (0, 0)),
            pl.BlockSpec((256, 512), lambda t: (0, 0)),
            pl.BlockSpec((512, 512), lambda t: (0, 0)),
            pl.BlockSpec((512, 64), lambda t: (0, 0)),
            pl.BlockSpec((512, 64), lambda t: (0, 0)),
            pl.BlockSpec((BM, N),
                         lambda t: (jnp.clip(t - 1, 0, NB - 1), 0)),
        ],
        out_specs=pl.BlockSpec((N, 64), lambda t: (0, 0)),
        out_shape=jax.ShapeDtypeStruct((N, 64), f32),
        scratch_shapes=[
            pltpu.VMEM((N, 256), BF),
            pltpu.VMEM((N, 256), BF),
            pltpu.VMEM((N, 512), BF),
            pltpu.VMEM((N, 512), BF),
            pltpu.VMEM((N, 512), BF),
            pltpu.VMEM((N, 512), BF),
            pltpu.VMEM((N, N), BF),
            pltpu.VMEM((BM, N), f32),
        ],
    )(x, Wp0, Wg0, Wl0, Wp1, Wg1, Wl1, Wp2, Wg2, Wl2, adj)


# speedup vs baseline: 1.1131x; 1.0008x over previous
import jax
import jax.numpy as jnp
from jax.experimental import pallas as pl
from jax.experimental.pallas import tpu as pltpu

N = 2048
BM = 256
NB = N // BM

BF = jnp.bfloat16

LAYERS = ((256, 256, True), (256, 512, True), (512, 64, False))


def _leaky(x):
    return jnp.where(x >= 0, x, 0.01 * x)


def _dot(a, b):
    return jnp.dot(a, b, preferred_element_type=jnp.float32)


def _body(x_ref, wp0_ref, wg0_ref, wl0_ref, wp1_ref, wg1_ref, wl1_ref,
          wp2_ref, wg2_ref, wl2_ref, adj_ref, o_ref,
          xb_scr, h1_scr, h2_scr, q_scr, v_scr, u_scr, adjb_scr, lg_scr):
    t = pl.program_id(0)

    w = ((wp0_ref, wg0_ref, wl0_ref), (wp1_ref, wg1_ref, wl1_ref),
         (wp2_ref, wg2_ref, wl2_ref))
    hs = (xb_scr, h1_scr, h2_scr)

    @pl.when((t >= 1) & (t <= NB))
    def _cache():
        adjb_scr[pl.ds((t - 1) * BM, BM), :] = adj_ref[...].astype(BF)

    def prep(l):
        h = hs[l][...]
        wp, wg, wl = w[l]
        cin, cout, _ = LAYERS[l]
        q_scr[:, :cin] = _dot(h, wp[...].astype(BF)).astype(BF)
        v_scr[:, :cout] = _dot(h, wg[...].astype(BF)).astype(BF)
        u_scr[:, :cout] = _dot(h, wl[...].astype(BF)).astype(BF)

    def logits(l, i):
        cin = LAYERS[l][0]
        q_i = q_scr[pl.ds(i * BM, BM), :cin]
        lg_scr[pl.ds((i % 2) * BM, BM), :] = jax.lax.dot_general(
            q_i, hs[l][...], (((1,), (1,)), ((), ())),
            preferred_element_type=jnp.float32)

    def finish(l, i):
        cout, act = LAYERS[l][1:]
        logits_i = lg_scr[pl.ds((i % 2) * BM, BM), :]
        m = jnp.max(logits_i, axis=1, keepdims=True)
        e = jnp.exp(logits_i - m)
        s = jnp.sum(e, axis=1, keepdims=True)
        og = _dot(e.astype(BF), v_scr[:, :cout]) / s
        ol = _dot(adjb_scr[pl.ds(i * BM, BM), :], u_scr[:, :cout])
        if act:
            out = _leaky(og) + _leaky(ol)
        else:
            out = og + ol
        if l == 2:
            o_ref[pl.ds(i * BM, BM), :] = out
        else:
            hs[l + 1][pl.ds(i * BM, BM), :] = out.astype(BF)

    @pl.when(t == 0)
    def _start():
        xb_scr[...] = x_ref[...].astype(BF)
        prep(0)
        logits(0, 0)

    for l in range(3):
        if l > 0:
            @pl.when(t == l * NB)
            def _boundary(l=l):
                finish(l - 1, NB - 1)
                prep(l)
                logits(l, 0)

        @pl.when((t > l * NB) & (t < (l + 1) * NB))
        def _steady(l=l):
            i = t - l * NB
            finish(l, i - 1)
            logits(l, i)

    @pl.when(t == 3 * NB)
    def _tail():
        finish(2, NB - 1)


def kernel(x, adj, Wp0, Wg0, Wl0, Wp1, Wg1, Wl1, Wp2, Wg2, Wl2):
    f32 = jnp.float32
    return pl.pallas_call(
        _body,
        grid=(3 * NB + 1,),
        in_specs=[
            pl.BlockSpec((N, 256), lambda t: (0, 0)),
            pl.BlockSpec((256, 256), lambda t: (0, 0)),
            pl.BlockSpec((256, 256), lambda t: (0, 0)),
            pl.BlockSpec((256, 256), lambda t: (0, 0)),
            pl.BlockSpec((256, 256), lambda t: (0, 0)),
            pl.BlockSpec((256, 512), lambda t: (0, 0)),
            pl.BlockSpec((256, 512), lambda t: (0, 0)),
            pl.BlockSpec((512, 512), lambda t: (0, 0)),
            pl.BlockSpec((512, 64), lambda t: (0, 0)),
            pl.BlockSpec((512, 64), lambda t: (0, 0)),
            pl.BlockSpec((BM, N),
                         lambda t: (jnp.clip(t - 1, 0, NB - 1), 0)),
        ],
        out_specs=pl.BlockSpec((N, 64), lambda t: (0, 0)),
        out_shape=jax.ShapeDtypeStruct((N, 64), f32),
        scratch_shapes=[
            pltpu.VMEM((N, 256), BF),
            pltpu.VMEM((N, 256), BF),
            pltpu.VMEM((N, 512), BF),
            pltpu.VMEM((N, 512), BF),
            pltpu.VMEM((N, 512), BF),
            pltpu.VMEM((N, 512), BF),
            pltpu.VMEM((N, N), BF),
            pltpu.VMEM((2 * BM, N), f32),
        ],
    )(x, Wp0, Wg0, Wl0, Wp1, Wg1, Wl1, Wp2, Wg2, Wl2, adj)
